# Initial kernel scaffold; baseline (speedup 1.0000x reference)
import jax
import jax.numpy as jnp
from jax.experimental import pallas as pl


def kernel(x, positions, note_table, pos_table, pent_mask):
    raise NotImplementedError("write your pallas kernel here")


# speedup vs baseline: 5.6989x; 5.6989x over previous
import functools

import jax
import jax.numpy as jnp
from jax import lax
from jax.experimental import pallas as pl
from jax.experimental.pallas import tpu as pltpu
from jax.experimental.pallas import tpu_sc as plsc

B, S, V, P, D = 4096, 200, 128, 512, 256
N = B * S
NC, NS, L = 2, 16, 16
NW = NC * NS
ROWS_PER_W = N // NW
C = 128
CHUNKS = ROWS_PER_W // C


def _fuse_note_table(note_table, pent_mask_col):
    def body(nt_ref, pm_ref, out_ref):
        out_ref[...] = nt_ref[...] + pm_ref[...] * 0.1

    return pl.pallas_call(
        body,
        out_shape=jax.ShapeDtypeStruct((V, D), jnp.float32),
    )(note_table, pent_mask_col)


@functools.partial(
    pl.kernel,
    out_type=jax.ShapeDtypeStruct((N, D), jnp.float32),
    mesh=plsc.VectorSubcoreMesh(core_axis_name="c", subcore_axis_name="s"),
    scratch_types=[
        pltpu.VMEM((C,), jnp.int32),
        pltpu.VMEM((C,), jnp.int32),
        pltpu.VMEM((C, D), jnp.float32),
        pltpu.VMEM((C, D), jnp.float32),
        pltpu.SemaphoreType.DMA,
        pltpu.SemaphoreType.DMA,
    ],
)
def _sc_lookup(note_hbm, pos_hbm, x_hbm, p_hbm, out_hbm,
               xi_v, pi_v, nrow_v, prow_v, sem_n, sem_p):
    wid = lax.axis_index("s") * NC + lax.axis_index("c")
    base = wid * ROWS_PER_W

    def chunk_body(ci, _):
        off = base + ci * C
        pltpu.sync_copy(x_hbm.at[pl.ds(off, C)], xi_v)
        pltpu.sync_copy(p_hbm.at[pl.ds(off, C)], pi_v)
        cp_n = pltpu.async_copy(note_hbm.at[xi_v], nrow_v, sem_n)
        cp_p = pltpu.async_copy(pos_hbm.at[pi_v], prow_v, sem_p)
        cp_n.wait()
        cp_p.wait()

        def add_body(i, _):
            r = i // (D // L)
            j = i % (D // L)
            sl = pl.ds(j * L, L)
            nrow_v[r, sl] = nrow_v[r, sl] + prow_v[r, sl]
            return ()

        lax.fori_loop(0, C * (D // L), add_body, ())
        pltpu.sync_copy(nrow_v, out_hbm.at[pl.ds(off, C)])
        return ()

    lax.fori_loop(0, CHUNKS, chunk_body, ())


def kernel(x, positions, note_table, pos_table, pent_mask):
    fused_note = _fuse_note_table(note_table, jnp.tile(pent_mask[:, None], (1, D)))
    out = _sc_lookup(fused_note, pos_table,
                     x.reshape(N), positions.reshape(N))
    return out.reshape(B, S, D)
